# baseline (speedup 1.0000x reference)
import functools

import jax
import jax.numpy as jnp
from jax import lax
from jax.experimental import pallas as pl
from jax.experimental.pallas import tpu as pltpu
from jax.experimental.pallas import tpu_sc as plsc

_KAPPA_MAX = 0.7
_N0 = 10.0

_SC_S = 2
_BN = 512
_CH = 16


def _sc_gather_matvec(patch_ids, theta, n_eff, Hflat, w, s_tc):
    N = patch_ids.shape[0]
    R, D = Hflat.shape
    info = plsc.get_sparse_core_info()
    NC, NS = info.num_cores, info.num_subcores
    NW = NC * NS
    bpw = N // NW
    rows_sc = R - s_tc * N
    rpw = rows_sc // NW
    nch = rpw // _CH
    row0_all = s_tc * N
    mesh = plsc.VectorSubcoreMesh(core_axis_name="c", subcore_axis_name="s")

    @functools.partial(
        pl.kernel,
        mesh=mesh,
        out_type=[
            jax.ShapeDtypeStruct((N,), jnp.float32),
            jax.ShapeDtypeStruct((N,), jnp.float32),
            jax.ShapeDtypeStruct((rows_sc,), jnp.float32),
        ],
        scratch_types=[
            pltpu.VMEM((bpw,), jnp.int32),
            pltpu.VMEM((bpw,), jnp.float32),
            pltpu.VMEM((bpw,), jnp.float32),
            pltpu.VMEM((D,), jnp.float32),
            pltpu.VMEM((_CH, D), jnp.float32),
            pltpu.VMEM((_CH, D), jnp.float32),
            pltpu.VMEM((rpw,), jnp.float32),
            pltpu.SemaphoreType.DMA,
            pltpu.SemaphoreType.DMA,
            pltpu.SemaphoreType.DMA,
            pltpu.SemaphoreType.DMA,
        ],
    )
    def gk(idx_hbm, theta_hbm, neff_hbm, h_hbm, w_hbm, th_out, ne_out, raw_out,
           idx_v, th_v, ne_v, w_v, buf0, buf1, out_v, s0, s1, s2, s3):
        wid = lax.axis_index("s") * NC + lax.axis_index("c")

        gbase = wid * bpw
        pltpu.sync_copy(idx_hbm.at[pl.ds(gbase, bpw)], idx_v)
        c1 = pltpu.async_copy(theta_hbm.at[idx_v], th_v, s0)
        c2 = pltpu.async_copy(neff_hbm.at[idx_v], ne_v, s1)

        pltpu.sync_copy(w_hbm, w_v)
        row0 = row0_all + wid * rpw
        bufs = (buf0, buf1)
        sems = (s2, s3)

        def start(c):
            return pltpu.async_copy(
                h_hbm.at[pl.ds(row0 + c * _CH, _CH)], bufs[c % 2], sems[c % 2])

        cps = [start(0)]
        if nch > 1:
            cps.append(start(1))
        c1.wait()
        c2.wait()
        pltpu.sync_copy(th_v, th_out.at[pl.ds(gbase, bpw)])
        pltpu.sync_copy(ne_v, ne_out.at[pl.ds(gbase, bpw)])

        for c in range(nch):
            buf = bufs[c % 2]
            cps[c].wait()

            def fma(k, accs):
                wk = w_v[pl.ds(k * 16, 16)]
                return tuple(
                    accs[j] + buf[j, pl.ds(k * 16, 16)] * wk
                    for j in range(_CH))

            accs = lax.fori_loop(
                0, D // 16, fma,
                tuple(jnp.zeros((16,), jnp.float32) for _ in range(_CH)))
            lane = lax.iota(jnp.int32, 16)
            rowsum = jnp.zeros((16,), jnp.float32)
            for j in range(_CH):
                t = accs[j]
                for d in (8, 4, 2, 1):
                    t = t + t.at[lane ^ d].get(mode="promise_in_bounds")
                rowsum = jnp.where(lane == j, t, rowsum)
            out_v[pl.ds(c * _CH, _CH)] = rowsum

            if c + 2 < nch:
                cps.append(start(c + 2))

        pltpu.sync_copy(out_v, raw_out.at[pl.ds(wid * rpw, rpw)])

    return gk(patch_ids, theta, n_eff, Hflat, w)


def _mv_body(h_ref, w_ref, b_ref, raw_ref):
    w = w_ref[...]
    h = h_ref[...]
    raw_ref[...] = jnp.sum(h * w[None], axis=-1) + b_ref[0, 0]


def _comb_body(s_tc, ra_ref, rb_ref, th_ref, ne_ref, b_ref, lt_ref,
               probs_ref, ls_ref):
    th = th_ref[...]
    ne = ne_ref[...]
    bl = jnp.log(th) - jnp.log(1.0 - th)
    kap = jnp.clip(_KAPPA_MAX * (_N0 / (ne + _N0)), 0.0, _KAPPA_MAX)
    t = jnp.log(1.0 + jnp.exp(lt_ref[0, 0])) + 1e-4

    ra = ra_ref[...]
    rb = rb_ref[...] + b_ref[0, 0]
    ls_a = (1.0 - kap) * ra + kap * bl
    ls_b = (1.0 - kap) * rb + kap * bl
    probs_ref[:s_tc] = 1.0 / (1.0 + jnp.exp(-ls_a / t))
    probs_ref[s_tc:] = 1.0 / (1.0 + jnp.exp(-ls_b / t))
    ls_ref[:s_tc] = ls_a
    ls_ref[s_tc:] = ls_b


def kernel(H_t, patch_ids, theta, n_eff, W, b, log_temperature):
    S, N, D = H_t.shape
    pid = patch_ids.astype(jnp.int32)
    s_tc = S - _SC_S
    Hflat = H_t.reshape(S * N, D)

    th_n, ne_n, raw_sc = _sc_gather_matvec(
        pid, theta, n_eff, Hflat, W.reshape(D), s_tc)

    BN = _BN
    raw_tc = pl.pallas_call(
        _mv_body,
        grid=(N // BN,),
        in_specs=[
            pl.BlockSpec((s_tc, BN, D), lambda i: (0, i, 0)),
            pl.BlockSpec((1, D), lambda i: (0, 0)),
            pl.BlockSpec(memory_space=pltpu.SMEM),
        ],
        out_specs=pl.BlockSpec((s_tc, BN), lambda i: (0, i)),
        out_shape=jax.ShapeDtypeStruct((s_tc, N), jnp.float32),
        compiler_params=pltpu.CompilerParams(
            dimension_semantics=("arbitrary",),
        ),
    )(H_t, W, b.reshape(1, 1))

    probs, ls = pl.pallas_call(
        functools.partial(_comb_body, s_tc),
        in_specs=[
            pl.BlockSpec((s_tc, N), lambda: (0, 0)),
            pl.BlockSpec((_SC_S, N), lambda: (0, 0)),
            pl.BlockSpec((1, N), lambda: (0, 0)),
            pl.BlockSpec((1, N), lambda: (0, 0)),
            pl.BlockSpec(memory_space=pltpu.SMEM),
            pl.BlockSpec(memory_space=pltpu.SMEM),
        ],
        out_specs=[
            pl.BlockSpec((S, N), lambda: (0, 0)),
            pl.BlockSpec((S, N), lambda: (0, 0)),
        ],
        out_shape=[
            jax.ShapeDtypeStruct((S, N), jnp.float32),
            jax.ShapeDtypeStruct((S, N), jnp.float32),
        ],
    )(raw_tc, raw_sc.reshape(_SC_S, N), th_n.reshape(1, N),
      ne_n.reshape(1, N), b.reshape(1, 1),
      log_temperature.astype(jnp.float32).reshape(1, 1))
    return probs, ls


# speedup vs baseline: 1.2330x; 1.2330x over previous
import functools

import jax
import jax.numpy as jnp
from jax import lax
from jax.experimental import pallas as pl
from jax.experimental.pallas import tpu as pltpu
from jax.experimental.pallas import tpu_sc as plsc

_KAPPA_MAX = 0.7
_N0 = 10.0

_BN = 512


def _sc_gather(patch_ids, theta, n_eff):
    N = patch_ids.shape[0]
    info = plsc.get_sparse_core_info()
    NC, NS = info.num_cores, info.num_subcores
    NW = NC * NS
    bpw = N // NW
    mesh = plsc.VectorSubcoreMesh(core_axis_name="c", subcore_axis_name="s")

    @functools.partial(
        pl.kernel,
        mesh=mesh,
        out_type=[
            jax.ShapeDtypeStruct((N,), jnp.float32),
            jax.ShapeDtypeStruct((N,), jnp.float32),
        ],
        scratch_types=[
            pltpu.VMEM((bpw,), jnp.int32),
            pltpu.VMEM((bpw,), jnp.float32),
            pltpu.VMEM((bpw,), jnp.float32),
            pltpu.SemaphoreType.DMA,
            pltpu.SemaphoreType.DMA,
        ],
    )
    def gk(idx_hbm, theta_hbm, neff_hbm, th_out, ne_out, idx_v, th_v, ne_v, s1, s2):
        wid = lax.axis_index("s") * NC + lax.axis_index("c")
        base = wid * bpw
        pltpu.sync_copy(idx_hbm.at[pl.ds(base, bpw)], idx_v)
        c1 = pltpu.async_copy(theta_hbm.at[idx_v], th_v, s1)
        c2 = pltpu.async_copy(neff_hbm.at[idx_v], ne_v, s2)
        c1.wait()
        c2.wait()
        pltpu.sync_copy(th_v, th_out.at[pl.ds(base, bpw)])
        pltpu.sync_copy(ne_v, ne_out.at[pl.ds(base, bpw)])

    return gk(patch_ids, theta, n_eff)


def _mv_body(h_ref, w_ref, b_ref, raw_ref):
    w = w_ref[...]
    h = h_ref[...]
    raw_ref[...] = jnp.sum(h * w[None], axis=-1) + b_ref[0, 0]


def _comb_body(raw_ref, th_ref, ne_ref, lt_ref, probs_ref, ls_ref):
    raw = raw_ref[...]
    th = th_ref[...]
    ne = ne_ref[...]
    bl = jnp.log(th) - jnp.log(1.0 - th)
    kap = jnp.clip(_KAPPA_MAX * (_N0 / (ne + _N0)), 0.0, _KAPPA_MAX)
    ls = (1.0 - kap) * raw + kap * bl
    t = jnp.log(1.0 + jnp.exp(lt_ref[0, 0])) + 1e-4
    probs_ref[...] = 1.0 / (1.0 + jnp.exp(-ls / t))
    ls_ref[...] = ls


def kernel(H_t, patch_ids, theta, n_eff, W, b, log_temperature):
    S, N, D = H_t.shape
    pid = patch_ids.astype(jnp.int32)
    th_n, ne_n = _sc_gather(pid, theta, n_eff)

    BN = _BN
    raw = pl.pallas_call(
        _mv_body,
        grid=(N // BN,),
        in_specs=[
            pl.BlockSpec((S, BN, D), lambda i: (0, i, 0)),
            pl.BlockSpec((1, D), lambda i: (0, 0)),
            pl.BlockSpec(memory_space=pltpu.SMEM),
        ],
        out_specs=pl.BlockSpec((S, BN), lambda i: (0, i)),
        out_shape=jax.ShapeDtypeStruct((S, N), jnp.float32),
        compiler_params=pltpu.CompilerParams(
            dimension_semantics=("arbitrary",),
        ),
    )(H_t, W, b.reshape(1, 1))

    probs, ls = pl.pallas_call(
        _comb_body,
        in_specs=[
            pl.BlockSpec((S, N), lambda: (0, 0)),
            pl.BlockSpec((1, N), lambda: (0, 0)),
            pl.BlockSpec((1, N), lambda: (0, 0)),
            pl.BlockSpec(memory_space=pltpu.SMEM),
        ],
        out_specs=[
            pl.BlockSpec((S, N), lambda: (0, 0)),
            pl.BlockSpec((S, N), lambda: (0, 0)),
        ],
        out_shape=[
            jax.ShapeDtypeStruct((S, N), jnp.float32),
            jax.ShapeDtypeStruct((S, N), jnp.float32),
        ],
    )(raw, th_n.reshape(1, N), ne_n.reshape(1, N),
      log_temperature.astype(jnp.float32).reshape(1, 1))
    return probs, ls
